# Initial kernel scaffold; baseline (speedup 1.0000x reference)
import jax
import jax.numpy as jnp
from jax.experimental import pallas as pl


def kernel(x, edge_index, batch, W1, a_src1, a_dst1, b1, W2, a_src2, a_dst2, b2, W3, a_src3, a_dst3, b3, align1_W, align1_b, align2_W, align2_b, ln1_g, ln1_b, ln2_g, ln2_b, ln3_g, ln3_b):
    raise NotImplementedError("write your pallas kernel here")


# speedup vs baseline: 1.0659x; 1.0659x over previous
import functools
import jax
import jax.numpy as jnp
import numpy as np
from jax.experimental import pallas as pl

_N = 50000
_EPS = 1e-16


def _transform_body(x_ref, w_ref, as_ref, ad_ref, xs_ref, als_ref, ald_ref):
    xs = jnp.dot(x_ref[...], w_ref[...], preferred_element_type=jnp.float32)
    xs_ref[...] = xs
    als_ref[...] = jnp.dot(xs, as_ref[...], preferred_element_type=jnp.float32)
    ald_ref[...] = jnp.dot(xs, ad_ref[...], preferred_element_type=jnp.float32)


def _node_transform(x, W, As, Ad, bn):
    n, din = x.shape
    d = W.shape[1]
    h = As.shape[1]
    grid = n // bn
    return pl.pallas_call(
        _transform_body,
        grid=(grid,),
        in_specs=[
            pl.BlockSpec((bn, din), lambda i: (i, 0)),
            pl.BlockSpec((din, d), lambda i: (0, 0)),
            pl.BlockSpec((d, h), lambda i: (0, 0)),
            pl.BlockSpec((d, h), lambda i: (0, 0)),
        ],
        out_specs=[
            pl.BlockSpec((bn, d), lambda i: (i, 0)),
            pl.BlockSpec((bn, h), lambda i: (i, 0)),
            pl.BlockSpec((bn, h), lambda i: (i, 0)),
        ],
        out_shape=[
            jax.ShapeDtypeStruct((n, d), jnp.float32),
            jax.ShapeDtypeStruct((n, h), jnp.float32),
            jax.ShapeDtypeStruct((n, h), jnp.float32),
        ],
    )(x, W, As, Ad)


def _post_body(num_ref, s_ref, rep_ref, b_ref, g_ref, be_ref, res_ref, out_ref):
    s_rep = jnp.dot(s_ref[...], rep_ref[...], preferred_element_type=jnp.float32)
    o = num_ref[...] / (s_rep + _EPS) + b_ref[...]
    o = jnp.where(o > 0, o, jnp.exp(o) - 1.0)
    mu = jnp.mean(o, axis=-1, keepdims=True)
    var = jnp.mean((o - mu) ** 2, axis=-1, keepdims=True)
    o = (o - mu) / jnp.sqrt(var + 1e-5) * g_ref[...] + be_ref[...]
    out_ref[...] = o + res_ref[...]


def _post_layer(num, s, rep, b, g, be, res, bn):
    n, d = num.shape
    h = s.shape[1]
    grid = n // bn
    return pl.pallas_call(
        _post_body,
        grid=(grid,),
        in_specs=[
            pl.BlockSpec((bn, d), lambda i: (i, 0)),
            pl.BlockSpec((bn, h), lambda i: (i, 0)),
            pl.BlockSpec((h, d), lambda i: (0, 0)),
            pl.BlockSpec((1, d), lambda i: (0, 0)),
            pl.BlockSpec((1, d), lambda i: (0, 0)),
            pl.BlockSpec((1, d), lambda i: (0, 0)),
            pl.BlockSpec((bn, d), lambda i: (i, 0)),
        ],
        out_specs=pl.BlockSpec((bn, d), lambda i: (i, 0)),
        out_shape=jax.ShapeDtypeStruct((n, d), jnp.float32),
    )(num, s, rep, b, g, be, res)


def _final_body(e1_ref, e2_ref, e3_ref, out_ref):
    def softmax(v):
        m = jnp.max(v)
        ex = jnp.exp(v - m)
        return ex / jnp.sum(ex)

    f = (softmax(e1_ref[...]) + softmax(e2_ref[...]) + softmax(e3_ref[...])) / 3.0
    mn = jnp.min(f)
    mx = jnp.max(f)
    out_ref[...] = (f - mn) / (mx - mn) * 100.0


def _final_scores(e1, e2, e3):
    ne = e1.shape[0]
    rows = ne // 128
    shp = (rows, 128)
    out = pl.pallas_call(
        _final_body,
        out_shape=jax.ShapeDtypeStruct(shp, jnp.float32),
    )(e1.reshape(shp), e2.reshape(shp), e3.reshape(shp))
    return out.reshape(ne)


def _edge_phase(xs, al_s, al_d, src, dst, h, c):
    n = xs.shape[0]
    alpha = al_s[src] + al_d[dst]
    alpha = jnp.where(alpha > 0, alpha, 0.2 * alpha)
    m = jnp.max(alpha, axis=0)
    ev = jnp.exp(alpha - m)
    s = jax.ops.segment_sum(ev, dst, num_segments=n)
    xg = xs[src].reshape(-1, h, c)
    num = jax.ops.segment_sum(ev[:, :, None] * xg, dst, num_segments=n)
    e_mean = jnp.mean(ev / (s[dst] + _EPS), axis=1)
    return num.reshape(n, h * c), s, e_mean


def _make_proj(a_vec):
    _, hh, cc = a_vec.shape
    p = jnp.zeros((hh * cc, hh), jnp.float32)
    ii = jnp.arange(hh * cc)
    return p.at[ii, ii // cc].set(a_vec.reshape(-1))


def _make_rep(hh, cc):
    r = jnp.zeros((hh, hh * cc), jnp.float32)
    ii = jnp.arange(hh * cc)
    return r.at[ii // cc, ii].set(1.0)


def kernel(x, edge_index, batch, W1, a_src1, a_dst1, b1, W2, a_src2, a_dst2, b2,
           W3, a_src3, a_dst3, b3, align1_W, align1_b, align2_W, align2_b,
           ln1_g, ln1_b, ln2_g, ln2_b, ln3_g, ln3_b):
    n = x.shape[0]
    ne = edge_index.shape[1]
    loops = jnp.arange(n, dtype=edge_index.dtype)
    src = jnp.concatenate([edge_index[0], loops])
    dst = jnp.concatenate([edge_index[1], loops])

    bn = 2000

    As1, Ad1 = _make_proj(a_src1), _make_proj(a_dst1)
    xs1, als1, ald1 = _node_transform(x, W1, As1, Ad1, bn)
    num1, s1, em1 = _edge_phase(xs1, als1, ald1, src, dst, 16, 4)
    res1 = x @ align1_W + align1_b
    h1 = _post_layer(num1 + 0.0, s1, _make_rep(16, 4), b1.reshape(1, -1),
                     ln1_g.reshape(1, -1), ln1_b.reshape(1, -1), res1, bn)

    As2, Ad2 = _make_proj(a_src2), _make_proj(a_dst2)
    xs2, als2, ald2 = _node_transform(h1, W2, As2, Ad2, bn)
    num2, s2, em2 = _edge_phase(xs2, als2, ald2, src, dst, 8, 4)
    res2 = h1 @ align2_W + align2_b
    h2 = _post_layer(num2, s2, _make_rep(8, 4), b2.reshape(1, -1),
                     ln2_g.reshape(1, -1), ln2_b.reshape(1, -1), res2, bn)

    As3, Ad3 = _make_proj(a_src3), _make_proj(a_dst3)
    xs3, als3, ald3 = _node_transform(h2, W3, As3, Ad3, bn)
    num3, s3, em3 = _edge_phase(xs3, als3, ald3, src, dst, 4, 4)
    zero_res = jnp.zeros((n, 16), jnp.float32)
    h3 = _post_layer(num3, s3, _make_rep(4, 4), b3.reshape(1, -1),
                     ln3_g.reshape(1, -1), ln3_b.reshape(1, -1), zero_res, bn)

    scores = _final_scores(em1[:ne], em2[:ne], em3[:ne])
    return h3, scores
